# baseline (speedup 1.0000x reference)
import functools

import jax
import jax.numpy as jnp
from jax import lax
from jax.experimental import pallas as pl
from jax.experimental.pallas import tpu as pltpu
from jax.experimental.pallas import tpu_sc as plsc

_INFO = plsc.get_sparse_core_info()
_NC = _INFO.num_cores
_NS = _INFO.num_subcores
_NW = _NC * _NS


@functools.lru_cache(maxsize=None)
def _build_gather(V, D, B, S):
  JG = D // 8
  NSG = S // 8
  NBT = B // 128
  n_su = NSG * NBT
  assert n_su % _NW == 0
  su_per_w = n_su // _NW
  n_steps = su_per_w * 8
  assert n_steps % 2 == 0 and n_steps >= 4

  mesh = plsc.VectorSubcoreMesh(core_axis_name="c", subcore_axis_name="s")

  @functools.partial(
      pl.kernel,
      mesh=mesh,
      out_type=jax.ShapeDtypeStruct((S, JG, NBT, 8, 128), jnp.float32),
      scratch_types=[
          pltpu.VMEM((2, 8, 128), jnp.int32),
          pltpu.VMEM((2, 128, D), jnp.float32),
          pltpu.VMEM((2, JG, 8, 129), jnp.float32),
          pltpu.SemaphoreType.DMA((2,)),
          pltpu.SemaphoreType.DMA((2,)),
          pltpu.SemaphoreType.DMA((2,)),
      ],
      compiler_params=pltpu.CompilerParams(use_tc_tiling_on_sc=False,
                                           needs_layout_passes=False),
  )
  def gather_kernel(idx_hbm, table_hbm, out_hbm, xt, rowbuf, tilebuf,
                    xsem, gsem, wsem):
    wid = lax.axis_index("s") * _NC + lax.axis_index("c")
    su0 = wid * su_per_w

    tokvecs = [b0 * 16 + lax.iota(jnp.int32, 16) for b0 in range(8)]

    def su_coords(su_i):
      su = su0 + su_i
      return su // NBT, su % NBT

    def load_xtile(su_i, slot):
      sg, bt = su_coords(su_i)
      pltpu.make_async_copy(idx_hbm.at[sg, bt], xt.at[slot],
                            xsem.at[slot]).start()

    def wait_xtile(slot):
      pltpu.make_async_copy(idx_hbm.at[0, 0], xt.at[slot],
                            xsem.at[slot]).wait()

    def start_gather(t, p):
      su_i, ds = t // 8, t % 8
      pltpu.make_async_copy(table_hbm.at[xt.at[su_i % 2, ds]], rowbuf.at[p],
                            gsem.at[p]).start()

    def wait_gather(p):
      pltpu.make_async_copy(table_hbm.at[xt.at[0, 0]], rowbuf.at[p],
                            gsem.at[p]).wait()

    def start_write(t, p):
      su_i, ds = t // 8, t % 8
      sg, bt = su_coords(su_i)
      s = sg * 8 + ds
      pltpu.make_async_copy(tilebuf.at[p, :, :, pl.ds(0, 128)],
                            out_hbm.at[s, :, bt], wsem.at[p]).start()

    def wait_write(p):
      pltpu.make_async_copy(tilebuf.at[p, :, :, pl.ds(0, 128)],
                            out_hbm.at[0, :, 0], wsem.at[p]).wait()

    lane = lax.iota(jnp.int32, 16)
    jg_vecs = [(j0 + lane) // 8 for j0 in range(0, D, 16)]
    jr_vecs = [(j0 + lane) % 8 for j0 in range(0, D, 16)]

    def transpose_block(p):
      rb = rowbuf.at[p]
      tb = tilebuf.at[p]
      for t0 in range(0, 128, 4):
        toks = range(t0, t0 + 4)
        vs = [rb[tok, pl.ds(j0, 16)]
              for tok in toks for j0 in range(0, D, 16)]
        for i, (tok, q) in enumerate(
            (tok, q) for tok in toks for q in range(D // 16)):
          plsc.store_scatter(
              tb, [jg_vecs[q], jr_vecs[q], jnp.full((16,), tok, jnp.int32)],
              vs[i])

    load_xtile(0, 0)
    wait_xtile(0)
    if su_per_w > 1:
      load_xtile(1, 1)
    start_gather(0, 0)
    start_gather(1, 1)

    def step(t, p):
      su_i, ds = t // 8, t % 8
      wait_gather(p)

      @pl.when(jnp.logical_and(ds == 7, su_i + 2 < su_per_w))
      def _():
        load_xtile(su_i + 2, su_i % 2)

      @pl.when(t >= 2)
      def _():
        wait_write(p)

      transpose_block(p)
      start_write(t, p)

      tn = t + 2
      @pl.when(tn < n_steps)
      def _():
        sun, dsn = tn // 8, tn % 8

        @pl.when(jnp.logical_and(dsn == 0, sun > 0))
        def _():
          wait_xtile(sun % 2)

        start_gather(tn, p)

    def lap(th, carry):
      t = th * 2
      step(t, 0)
      step(t + 1, 1)
      return carry

    lax.fori_loop(0, n_steps // 2, lap, 0)
    wait_write(0)
    wait_write(1)

  return gather_kernel


def kernel(x, table):
  B, S = x.shape
  V, D = table.shape
  assert B % 128 == 0 and S % 8 == 0 and D % 8 == 0

  idx = (x.astype(jnp.int32)
         .reshape(B // 128, 128, S // 8, 8)
         .transpose(2, 0, 3, 1))

  out5 = _build_gather(V, D, B, S)(idx, table)
  out = out5.transpose(2, 4, 0, 1, 3).reshape(B, S, D)
  return out


# speedup vs baseline: 1.0419x; 1.0419x over previous
import functools

import jax
import jax.numpy as jnp
from jax import lax
from jax.experimental import pallas as pl
from jax.experimental.pallas import tpu as pltpu
from jax.experimental.pallas import tpu_sc as plsc

_INFO = plsc.get_sparse_core_info()
_NC = _INFO.num_cores
_NS = _INFO.num_subcores
_NW = _NC * _NS


@functools.lru_cache(maxsize=None)
def _build_gather(V, D, B, S):
  JG = D // 8
  NSG = S // 8
  NBT = B // 128
  n_su = NSG * NBT
  assert n_su % _NW == 0
  su_per_w = n_su // _NW
  n_steps = su_per_w * 8
  assert n_steps % 2 == 0 and n_steps >= 4

  mesh = plsc.VectorSubcoreMesh(core_axis_name="c", subcore_axis_name="s")

  @functools.partial(
      pl.kernel,
      mesh=mesh,
      out_type=jax.ShapeDtypeStruct((S, JG, NBT, 8, 128), jnp.float32),
      scratch_types=[
          pltpu.VMEM((2, 8, 128), jnp.int32),
          pltpu.VMEM((2, 128, D), jnp.float32),
          pltpu.VMEM((2, JG, 8, 129), jnp.float32),
          pltpu.SemaphoreType.DMA((2,)),
          pltpu.SemaphoreType.DMA((2,)),
          pltpu.SemaphoreType.DMA((2,)),
      ],
      compiler_params=pltpu.CompilerParams(use_tc_tiling_on_sc=False,
                                           needs_layout_passes=False),
  )
  def gather_kernel(idx_hbm, table_hbm, out_hbm, xt, rowbuf, tilebuf,
                    xsem, gsem, wsem):
    wid = lax.axis_index("s") * _NC + lax.axis_index("c")
    su0 = wid * su_per_w

    tokvecs = [b0 * 16 + lax.iota(jnp.int32, 16) for b0 in range(8)]

    def su_coords(su_i):
      su = su0 + su_i
      return su // NBT, su % NBT

    def load_xtile(su_i, slot):
      sg, bt = su_coords(su_i)
      pltpu.make_async_copy(idx_hbm.at[sg, bt], xt.at[slot],
                            xsem.at[slot]).start()

    def wait_xtile(slot):
      pltpu.make_async_copy(idx_hbm.at[0, 0], xt.at[slot],
                            xsem.at[slot]).wait()

    def start_gather(t, p):
      su_i, ds = t // 8, t % 8
      pltpu.make_async_copy(table_hbm.at[xt.at[su_i % 2, ds]], rowbuf.at[p],
                            gsem.at[p]).start()

    def wait_gather(p):
      pltpu.make_async_copy(table_hbm.at[xt.at[0, 0]], rowbuf.at[p],
                            gsem.at[p]).wait()

    def start_write(t, p):
      su_i, ds = t // 8, t % 8
      sg, bt = su_coords(su_i)
      s = sg * 8 + ds
      pltpu.make_async_copy(tilebuf.at[p, :, :, pl.ds(0, 128)],
                            out_hbm.at[s, :, bt], wsem.at[p]).start()

    def wait_write(p):
      pltpu.make_async_copy(tilebuf.at[p, :, :, pl.ds(0, 128)],
                            out_hbm.at[0, :, 0], wsem.at[p]).wait()

    lane = lax.iota(jnp.int32, 16)
    jg_vecs = [(j0 + lane) // 8 for j0 in range(0, D, 16)]
    jr_vecs = [(j0 + lane) % 8 for j0 in range(0, D, 16)]

    def transpose_block(p):
      rb = rowbuf.at[p]
      tb = tilebuf.at[p]
      for t0 in range(0, 128, 2):
        toks = range(t0, t0 + 2)
        vs = [rb[tok, pl.ds(j0, 16)]
              for tok in toks for j0 in range(0, D, 16)]
        for i, (tok, q) in enumerate(
            (tok, q) for tok in toks for q in range(D // 16)):
          plsc.store_scatter(
              tb, [jg_vecs[q], jr_vecs[q], jnp.full((16,), tok, jnp.int32)],
              vs[i])

    load_xtile(0, 0)
    wait_xtile(0)
    if su_per_w > 1:
      load_xtile(1, 1)
    start_gather(0, 0)
    start_gather(1, 1)

    def step(t, p):
      su_i, ds = t // 8, t % 8
      wait_gather(p)

      @pl.when(jnp.logical_and(ds == 7, su_i + 2 < su_per_w))
      def _():
        load_xtile(su_i + 2, su_i % 2)

      @pl.when(t >= 2)
      def _():
        wait_write(p)

      transpose_block(p)
      start_write(t, p)

      tn = t + 2
      @pl.when(tn < n_steps)
      def _():
        sun, dsn = tn // 8, tn % 8

        @pl.when(jnp.logical_and(dsn == 0, sun > 0))
        def _():
          wait_xtile(sun % 2)

        start_gather(tn, p)

    def lap(th, carry):
      t = th * 2
      step(t, 0)
      step(t + 1, 1)
      return carry

    lax.fori_loop(0, n_steps // 2, lap, 0)
    wait_write(0)
    wait_write(1)

  return gather_kernel


def kernel(x, table):
  B, S = x.shape
  V, D = table.shape
  assert B % 128 == 0 and S % 8 == 0 and D % 8 == 0

  idx = (x.astype(jnp.int32)
         .reshape(B // 128, 128, S // 8, 8)
         .transpose(2, 0, 3, 1))

  out5 = _build_gather(V, D, B, S)(idx, table)
  out = out5.transpose(2, 4, 0, 1, 3).reshape(B, S, D)
  return out
